# baseline (speedup 1.0000x reference)
import functools

import jax
import jax.numpy as jnp
from jax import lax
from jax.experimental import pallas as pl
from jax.experimental.pallas import tpu as pltpu
from jax.experimental.pallas import tpu_sc as plsc

N = 10000
E = 320000
IN_DIM = 128
OUT_DIM = 128
HALF = OUT_DIM // 2
NC = 2
NS = 16
LANES = 16
BN = 1000
C = 80
EPS = E // NS
CH = EPS // C
RPT = N // NS
ZR = 125
NBUF = 5


def _h_body(w_ref, W_ref, b_ref, x_ref, oh_ref, out_ref):
    w0 = w_ref[0]
    w1 = w_ref[1]
    w2 = w_ref[2]
    w3 = w_ref[3]
    Wc = w0 * W_ref[0] + w1 * W_ref[1] + w2 * W_ref[2]
    bc = w0 * b_ref[0] + w1 * b_ref[1] + w2 * b_ref[2]
    h = jnp.dot(x_ref[...], Wc, preferred_element_type=jnp.float32)
    h = h + bc[None, :] + w3 * oh_ref[...]
    out_ref[0] = h[:, :HALF]
    out_ref[1] = h[:, HALF:]


def _compute_h(weights, W, b, x, one_hot_h):
    return pl.pallas_call(
        _h_body,
        grid=(N // BN,),
        in_specs=[
            pl.BlockSpec(memory_space=pltpu.SMEM),
            pl.BlockSpec((NC + 1, IN_DIM, OUT_DIM), lambda i: (0, 0, 0)),
            pl.BlockSpec((NC + 1, OUT_DIM), lambda i: (0, 0)),
            pl.BlockSpec((BN, IN_DIM), lambda i: (i, 0)),
            pl.BlockSpec((BN, OUT_DIM), lambda i: (i, 0)),
        ],
        out_specs=pl.BlockSpec((2, BN, HALF), lambda i: (0, i, 0)),
        out_shape=jax.ShapeDtypeStruct((2, N, HALF), jnp.float32),
    )(weights, W, b, x, one_hot_h)


def _spmm_body(src_hbm, dst_hbm, vals_hbm, h_hbm, out_hbm,
               gidx, didx, vals_v, rows,
               g0, g1, g2, g3, g4, s0, s1, s2, s3, s4, accum):
    gsems = (g0, g1, g2, g3, g4)
    ssems = (s0, s1, s2, s3, s4)
    c = lax.axis_index("c")
    s = lax.axis_index("s")

    pltpu.sync_copy(src_hbm.at[s], gidx)
    pltpu.sync_copy(dst_hbm.at[s], didx)
    pltpu.sync_copy(vals_hbm.at[s], vals_v)

    cN = c * N

    @pl.loop(0, CH)
    def _(r):
        for j in range(C // LANES):
            sl = pl.ds(j * LANES, LANES)
            gidx[r, sl] = gidx[r, sl] + cN

    @pl.loop(0, C)
    def _(r):
        for b in range(NBUF):
            for j in range(HALF // LANES):
                rows[b, r, pl.ds(j * LANES, LANES)] = jnp.zeros(
                    (LANES,), jnp.float32)

    for i in range(RPT // C):
        pltpu.sync_copy(rows.at[i % NBUF],
                        accum.at[pl.ds(s * RPT + i * C, C)])
    _REM = RPT % C
    pltpu.sync_copy(rows.at[0].at[pl.ds(0, _REM)],
                    accum.at[pl.ds(s * RPT + (RPT // C) * C, _REM)])
    plsc.subcore_barrier()

    def wait_gather(kk, par):
        pltpu.make_async_copy(h_hbm.at[gidx.at[kk]], rows.at[par],
                              gsems[par]).wait()

    def fire_gather(kk, par):
        pltpu.async_copy(h_hbm.at[gidx.at[kk]], rows.at[par], gsems[par])

    def wait_scatter(kk, par):
        pltpu.make_async_copy(rows.at[par], accum.at[didx.at[kk]],
                              ssems[par]).wait()

    def fire_scatter(kk, par):
        pltpu.async_copy(rows.at[par], accum.at[didx.at[kk]], ssems[par],
                         add=True)

    def scale(kk, par):
        @pl.loop(0, C // LANES)
        def _(g):
            vv = vals_v[kk, pl.ds(g * LANES, LANES)]
            for l in range(LANES):
                e = g * LANES + l
                v = vv[l]
                for j in range(HALF // LANES):
                    sl = pl.ds(j * LANES, LANES)
                    rows[par, e, sl] = rows[par, e, sl] * v

    LEAD = NBUF - 1
    for i in range(LEAD):
        fire_gather(i, i)

    MAIN = (CH - LEAD) // NBUF * NBUF

    @pl.loop(0, MAIN, step=NBUF)
    def _(k):
        for par in range(NBUF):
            kk = k + par
            wait_gather(kk, par)
            nxt = (par + LEAD) % NBUF

            @pl.when(kk >= 1)
            def _():
                wait_scatter(kk - 1, nxt)

            fire_gather(kk + LEAD, nxt)
            scale(kk, par)
            fire_scatter(kk, par)

    for kk in range(MAIN, CH):
        par = kk % NBUF
        wait_gather(kk, par)
        nxt = (par + LEAD) % NBUF
        wait_scatter(kk - 1, nxt)
        if kk + LEAD < CH:
            fire_gather(kk + LEAD, nxt)
        scale(kk, par)
        fire_scatter(kk, par)

    wait_scatter(CH - 1, (CH - 1) % NBUF)

    plsc.subcore_barrier()
    pltpu.sync_copy(accum.at[pl.ds(s * RPT, RPT)],
                    out_hbm.at[pl.ds(s * RPT, RPT), pl.ds(c * HALF, HALF)])


@functools.cache
def _make_spmm():
    return pl.kernel(
        _spmm_body,
        out_type=jax.ShapeDtypeStruct((N, OUT_DIM), jnp.float32),
        mesh=plsc.VectorSubcoreMesh(core_axis_name="c", subcore_axis_name="s"),
        scratch_types=[
            pltpu.VMEM((CH, C), jnp.int32),
            pltpu.VMEM((CH, C), jnp.int32),
            pltpu.VMEM((CH, C), jnp.float32),
            pltpu.VMEM((NBUF, C, HALF), jnp.float32),
            pltpu.SemaphoreType.DMA,
            pltpu.SemaphoreType.DMA,
            pltpu.SemaphoreType.DMA,
            pltpu.SemaphoreType.DMA,
            pltpu.SemaphoreType.DMA,
            pltpu.SemaphoreType.DMA,
            pltpu.SemaphoreType.DMA,
            pltpu.SemaphoreType.DMA,
            pltpu.SemaphoreType.DMA,
            pltpu.SemaphoreType.DMA,
            pltpu.VMEM_SHARED((N, HALF), jnp.float32),
        ],
        compiler_params=pltpu.CompilerParams(use_tc_tiling_on_sc=False),
    )


@jax.jit
def kernel(edge_index, edge_vals, x, one_hot_h, weights, W, b):
    h2 = _compute_h(weights, W, b, x, one_hot_h).reshape(2 * N, HALF)
    src3 = edge_index[1].reshape(NS, CH, C)
    dst3 = edge_index[0].reshape(NS, CH, C)
    vals3 = edge_vals.reshape(NS, CH, C)
    return _make_spmm()(src3, dst3, vals3, h2)


# speedup vs baseline: 1.1889x; 1.1889x over previous
import functools

import jax
import jax.numpy as jnp
from jax import lax
from jax.experimental import pallas as pl
from jax.experimental.pallas import tpu as pltpu
from jax.experimental.pallas import tpu_sc as plsc

N = 10000
E = 320000
IN_DIM = 128
OUT_DIM = 128
HALF = OUT_DIM // 2
NC = 2
NS = 16
LANES = 16
BN = 1000
C = 96
EPS = E // NS
CH = -(-EPS // C)
PAD = CH * C - EPS
RPT = N // NS
ZR = 125
NBUF = 5


def _h_body(w_ref, W_ref, b_ref, x_ref, oh_ref, out_ref):
    w0 = w_ref[0]
    w1 = w_ref[1]
    w2 = w_ref[2]
    w3 = w_ref[3]
    Wc = w0 * W_ref[0] + w1 * W_ref[1] + w2 * W_ref[2]
    bc = w0 * b_ref[0] + w1 * b_ref[1] + w2 * b_ref[2]
    h = jnp.dot(x_ref[...], Wc, preferred_element_type=jnp.float32)
    h = h + bc[None, :] + w3 * oh_ref[...]
    out_ref[0] = h[:, :HALF]
    out_ref[1] = h[:, HALF:]


def _compute_h(weights, W, b, x, one_hot_h):
    return pl.pallas_call(
        _h_body,
        grid=(N // BN,),
        in_specs=[
            pl.BlockSpec(memory_space=pltpu.SMEM),
            pl.BlockSpec((NC + 1, IN_DIM, OUT_DIM), lambda i: (0, 0, 0)),
            pl.BlockSpec((NC + 1, OUT_DIM), lambda i: (0, 0)),
            pl.BlockSpec((BN, IN_DIM), lambda i: (i, 0)),
            pl.BlockSpec((BN, OUT_DIM), lambda i: (i, 0)),
        ],
        out_specs=pl.BlockSpec((2, BN, HALF), lambda i: (0, i, 0)),
        out_shape=jax.ShapeDtypeStruct((2, N, HALF), jnp.float32),
    )(weights, W, b, x, one_hot_h)


def _spmm_body(src_hbm, dst_hbm, vals_hbm, h_hbm, out_hbm,
               gidx, didx, vals_v, rows,
               g0, g1, g2, g3, g4, s0, s1, s2, s3, s4, accum):
    gsems = (g0, g1, g2, g3, g4)
    ssems = (s0, s1, s2, s3, s4)
    c = lax.axis_index("c")
    s = lax.axis_index("s")

    pltpu.sync_copy(src_hbm.at[s], gidx)
    pltpu.sync_copy(dst_hbm.at[s], didx)
    pltpu.sync_copy(vals_hbm.at[s], vals_v)

    cN = c * N

    @pl.loop(0, CH)
    def _(r):
        for j in range(C // LANES):
            sl = pl.ds(j * LANES, LANES)
            gidx[r, sl] = gidx[r, sl] + cN

    @pl.loop(0, C)
    def _(r):
        for b in range(NBUF):
            for j in range(HALF // LANES):
                rows[b, r, pl.ds(j * LANES, LANES)] = jnp.zeros(
                    (LANES,), jnp.float32)

    for i in range(RPT // C):
        pltpu.sync_copy(rows.at[i % NBUF],
                        accum.at[pl.ds(s * RPT + i * C, C)])
    _REM = RPT % C
    pltpu.sync_copy(rows.at[0].at[pl.ds(0, _REM)],
                    accum.at[pl.ds(s * RPT + (RPT // C) * C, _REM)])
    plsc.subcore_barrier()

    def wait_gather(kk, par):
        pltpu.make_async_copy(h_hbm.at[gidx.at[kk]], rows.at[par],
                              gsems[par]).wait()

    def fire_gather(kk, par):
        pltpu.async_copy(h_hbm.at[gidx.at[kk]], rows.at[par], gsems[par])

    def wait_scatter(kk, par):
        pltpu.make_async_copy(rows.at[par], accum.at[didx.at[kk]],
                              ssems[par]).wait()

    def fire_scatter(kk, par):
        pltpu.async_copy(rows.at[par], accum.at[didx.at[kk]], ssems[par],
                         add=True)

    def scale(kk, par):
        @pl.loop(0, C // LANES)
        def _(g):
            vv = vals_v[kk, pl.ds(g * LANES, LANES)]
            for l in range(LANES):
                e = g * LANES + l
                v = vv[l]
                for j in range(HALF // LANES):
                    sl = pl.ds(j * LANES, LANES)
                    rows[par, e, sl] = rows[par, e, sl] * v

    LEAD = NBUF - 2
    fire_gather(0, 0)
    fire_gather(1, 1)
    fire_gather(2, 2)

    MAIN = (CH - LEAD) // NBUF * NBUF

    @pl.loop(0, MAIN, step=NBUF)
    def _(k):
        for par in range(NBUF):
            kk = k + par
            wait_gather(kk, par)
            nxt = (par + LEAD) % NBUF

            @pl.when(kk >= 2)
            def _():
                wait_scatter(kk - 2, nxt)

            fire_gather(kk + LEAD, nxt)
            scale(kk, par)
            fire_scatter(kk, par)

    for kk in range(MAIN, CH):
        par = kk % NBUF
        wait_gather(kk, par)
        nxt = (par + LEAD) % NBUF
        wait_scatter(kk - 2, nxt)
        if kk + LEAD < CH:
            fire_gather(kk + LEAD, nxt)
        scale(kk, par)
        fire_scatter(kk, par)

    wait_scatter(CH - 2, (CH - 2) % NBUF)
    wait_scatter(CH - 1, (CH - 1) % NBUF)

    plsc.subcore_barrier()
    pltpu.sync_copy(accum.at[pl.ds(s * RPT, RPT)],
                    out_hbm.at[pl.ds(s * RPT, RPT), pl.ds(c * HALF, HALF)])


@functools.cache
def _make_spmm():
    return pl.kernel(
        _spmm_body,
        out_type=jax.ShapeDtypeStruct((N, OUT_DIM), jnp.float32),
        mesh=plsc.VectorSubcoreMesh(core_axis_name="c", subcore_axis_name="s"),
        scratch_types=[
            pltpu.VMEM((CH, C), jnp.int32),
            pltpu.VMEM((CH, C), jnp.int32),
            pltpu.VMEM((CH, C), jnp.float32),
            pltpu.VMEM((NBUF, C, HALF), jnp.float32),
            pltpu.SemaphoreType.DMA,
            pltpu.SemaphoreType.DMA,
            pltpu.SemaphoreType.DMA,
            pltpu.SemaphoreType.DMA,
            pltpu.SemaphoreType.DMA,
            pltpu.SemaphoreType.DMA,
            pltpu.SemaphoreType.DMA,
            pltpu.SemaphoreType.DMA,
            pltpu.SemaphoreType.DMA,
            pltpu.SemaphoreType.DMA,
            pltpu.VMEM_SHARED((N, HALF), jnp.float32),
        ],
        compiler_params=pltpu.CompilerParams(use_tc_tiling_on_sc=False),
    )


@jax.jit
def kernel(edge_index, edge_vals, x, one_hot_h, weights, W, b):
    h2 = _compute_h(weights, W, b, x, one_hot_h).reshape(2 * N, HALF)
    pad = ((0, 0), (0, PAD))
    src3 = jnp.pad(edge_index[1].reshape(NS, EPS), pad).reshape(NS, CH, C)
    dst3 = jnp.pad(edge_index[0].reshape(NS, EPS), pad).reshape(NS, CH, C)
    vals3 = jnp.pad(edge_vals.reshape(NS, EPS), pad).reshape(NS, CH, C)
    return _make_spmm()(src3, dst3, vals3, h2)


# speedup vs baseline: 1.3275x; 1.1166x over previous
import functools

import jax
import jax.numpy as jnp
from jax import lax
from jax.experimental import pallas as pl
from jax.experimental.pallas import tpu as pltpu
from jax.experimental.pallas import tpu_sc as plsc

N = 10000
E = 320000
IN_DIM = 128
OUT_DIM = 128
HALF = OUT_DIM // 2
NC = 2
NS = 16
LANES = 16
BN = 1000
C = 80
EPS = E // NS
CH = EPS // C
RPT = N // NS
ZR = 125
NBUF = 5


def _h_body(w_ref, W_ref, b_ref, x_ref, oh_ref, out_ref):
    w0 = w_ref[0]
    w1 = w_ref[1]
    w2 = w_ref[2]
    w3 = w_ref[3]
    Wc = w0 * W_ref[0] + w1 * W_ref[1] + w2 * W_ref[2]
    bc = w0 * b_ref[0] + w1 * b_ref[1] + w2 * b_ref[2]
    h = jnp.dot(x_ref[...], Wc, preferred_element_type=jnp.float32)
    h = h + bc[None, :] + w3 * oh_ref[...]
    out_ref[0] = h[:, :HALF]
    out_ref[1] = h[:, HALF:]


def _compute_h(weights, W, b, x, one_hot_h):
    return pl.pallas_call(
        _h_body,
        grid=(N // BN,),
        in_specs=[
            pl.BlockSpec(memory_space=pltpu.SMEM),
            pl.BlockSpec((NC + 1, IN_DIM, OUT_DIM), lambda i: (0, 0, 0)),
            pl.BlockSpec((NC + 1, OUT_DIM), lambda i: (0, 0)),
            pl.BlockSpec((BN, IN_DIM), lambda i: (i, 0)),
            pl.BlockSpec((BN, OUT_DIM), lambda i: (i, 0)),
        ],
        out_specs=pl.BlockSpec((2, BN, HALF), lambda i: (0, i, 0)),
        out_shape=jax.ShapeDtypeStruct((2, N, HALF), jnp.float32),
    )(weights, W, b, x, one_hot_h)


def _spmm_body(src_hbm, dst_hbm, vals_hbm, h_hbm, out_hbm,
               gidx, didx, vals_v, rows,
               g0, g1, g2, g3, g4, s0, s1, s2, s3, s4, accum):
    gsems = (g0, g1, g2, g3, g4)
    ssems = (s0, s1, s2, s3, s4)
    c = lax.axis_index("c")
    s = lax.axis_index("s")

    pltpu.sync_copy(src_hbm.at[s], gidx)
    pltpu.sync_copy(dst_hbm.at[s], didx)
    pltpu.sync_copy(vals_hbm.at[s], vals_v)

    cN = c * N

    @pl.loop(0, CH)
    def _(r):
        for j in range(C // LANES):
            sl = pl.ds(j * LANES, LANES)
            gidx[r, sl] = gidx[r, sl] + cN

    @pl.loop(0, C)
    def _(r):
        for b in range(NBUF):
            for j in range(HALF // LANES):
                rows[b, r, pl.ds(j * LANES, LANES)] = jnp.zeros(
                    (LANES,), jnp.float32)

    for i in range(RPT // C):
        pltpu.sync_copy(rows.at[i % NBUF],
                        accum.at[pl.ds(s * RPT + i * C, C)])
    _REM = RPT % C
    pltpu.sync_copy(rows.at[0].at[pl.ds(0, _REM)],
                    accum.at[pl.ds(s * RPT + (RPT // C) * C, _REM)])
    plsc.subcore_barrier()

    def wait_gather(kk, par):
        pltpu.make_async_copy(h_hbm.at[gidx.at[kk]], rows.at[par],
                              gsems[par]).wait()

    def fire_gather(kk, par):
        pltpu.async_copy(h_hbm.at[gidx.at[kk]], rows.at[par], gsems[par])

    def wait_scatter(kk, par):
        pltpu.make_async_copy(rows.at[par], accum.at[didx.at[kk]],
                              ssems[par]).wait()

    def fire_scatter(kk, par):
        pltpu.async_copy(rows.at[par], accum.at[didx.at[kk]], ssems[par],
                         add=True)

    def scale(kk, par):
        @pl.loop(0, C // LANES)
        def _(g):
            vv = vals_v[kk, pl.ds(g * LANES, LANES)]
            for l in range(LANES):
                e = g * LANES + l
                v = vv[l]
                for j in range(HALF // LANES):
                    sl = pl.ds(j * LANES, LANES)
                    rows[par, e, sl] = rows[par, e, sl] * v

    LEAD = NBUF - 2
    fire_gather(0, 0)
    fire_gather(1, 1)
    fire_gather(2, 2)

    MAIN = (CH - LEAD) // NBUF * NBUF

    @pl.loop(0, MAIN, step=NBUF)
    def _(k):
        for par in range(NBUF):
            kk = k + par
            wait_gather(kk, par)
            nxt = (par + LEAD) % NBUF

            @pl.when(kk >= 2)
            def _():
                wait_scatter(kk - 2, nxt)

            fire_gather(kk + LEAD, nxt)
            scale(kk, par)
            fire_scatter(kk, par)

    for kk in range(MAIN, CH):
        par = kk % NBUF
        wait_gather(kk, par)
        nxt = (par + LEAD) % NBUF
        wait_scatter(kk - 2, nxt)
        if kk + LEAD < CH:
            fire_gather(kk + LEAD, nxt)
        scale(kk, par)
        fire_scatter(kk, par)

    wait_scatter(CH - 2, (CH - 2) % NBUF)
    wait_scatter(CH - 1, (CH - 1) % NBUF)

    plsc.subcore_barrier()
    pltpu.sync_copy(accum.at[pl.ds(s * RPT, RPT)],
                    out_hbm.at[pl.ds(s * RPT, RPT), pl.ds(c * HALF, HALF)])


@functools.cache
def _make_spmm():
    return pl.kernel(
        _spmm_body,
        out_type=jax.ShapeDtypeStruct((N, OUT_DIM), jnp.float32),
        mesh=plsc.VectorSubcoreMesh(core_axis_name="c", subcore_axis_name="s"),
        scratch_types=[
            pltpu.VMEM((CH, C), jnp.int32),
            pltpu.VMEM((CH, C), jnp.int32),
            pltpu.VMEM((CH, C), jnp.float32),
            pltpu.VMEM((NBUF, C, HALF), jnp.float32),
            pltpu.SemaphoreType.DMA,
            pltpu.SemaphoreType.DMA,
            pltpu.SemaphoreType.DMA,
            pltpu.SemaphoreType.DMA,
            pltpu.SemaphoreType.DMA,
            pltpu.SemaphoreType.DMA,
            pltpu.SemaphoreType.DMA,
            pltpu.SemaphoreType.DMA,
            pltpu.SemaphoreType.DMA,
            pltpu.SemaphoreType.DMA,
            pltpu.VMEM_SHARED((N, HALF), jnp.float32),
        ],
        compiler_params=pltpu.CompilerParams(use_tc_tiling_on_sc=False),
    )


@jax.jit
def kernel(edge_index, edge_vals, x, one_hot_h, weights, W, b):
    h2 = _compute_h(weights, W, b, x, one_hot_h).reshape(2 * N, HALF)
    src3 = edge_index[1].reshape(NS, CH, C)
    dst3 = edge_index[0].reshape(NS, CH, C)
    vals3 = edge_vals.reshape(NS, CH, C)
    return _make_spmm()(src3, dst3, vals3, h2)
